# baseline (speedup 1.0000x reference)
import functools

import jax
import jax.numpy as jnp
from jax.experimental import pallas as pl
from jax.experimental.pallas import tpu as pltpu

T = 64
D = 768
E = 64
I2 = 1536
EPG = 2
LIMIT = 7.0
ALPHA = 1.702
BIG = 1 << 20

_DN = (((1,), (1,)), ((), ()))


def _gate_body(x_ref, nw_ref, gw_ref, gb_ref, t_ref, c_ref, ord_ref):
    xv = x_ref[...]
    ms = jnp.mean(xv * xv, axis=1, keepdims=True)
    eps = jnp.finfo(jnp.float32).eps
    t = xv * jax.lax.rsqrt(ms + eps) * nw_ref[...]
    t_ref[...] = t
    g = jax.lax.dot_general(t, gw_ref[...], _DN,
                            preferred_element_type=jnp.float32)
    g = g + gb_ref[...]
    col = jax.lax.broadcasted_iota(jnp.int32, (T, E), 1)
    m1 = jnp.max(g, axis=1, keepdims=True)
    i1 = jnp.min(jnp.where(g == m1, col, E), axis=1, keepdims=True)
    oh1 = col == i1
    gm = jnp.where(oh1, -jnp.inf, g)
    m2 = jnp.max(gm, axis=1, keepdims=True)
    i2 = jnp.min(jnp.where(gm == m2, col, E), axis=1, keepdims=True)
    oh2 = col == i2
    p1 = jax.nn.sigmoid(m1 - m2)
    c_ref[...] = jnp.where(oh1, p1, 0.0) + jnp.where(oh2, 1.0 - p1, 0.0)

    used = jnp.max((oh1 | oh2).astype(jnp.int32), axis=0, keepdims=True)
    erow = jax.lax.broadcasted_iota(jnp.int32, (E, E), 0)
    icol = jax.lax.broadcasted_iota(jnp.int32, (E, E), 1)
    lt = (erow <= icol).astype(jnp.float32)
    csum = jax.lax.dot_general(used.astype(jnp.float32), lt,
                               (((1,), (0,)), ((), ())),
                               preferred_element_type=jnp.float32)
    csum_i = csum.astype(jnp.int32)
    used_e = jnp.broadcast_to(used.reshape(E, 1), (E, E)) > 0
    csum_e = jnp.broadcast_to(csum_i.reshape(E, 1), (E, E))
    cond = used_e & (csum_e == icol + 1)
    order_raw = jnp.min(jnp.where(cond, erow, BIG), axis=0, keepdims=True)
    lane = jax.lax.broadcasted_iota(jnp.int32, (1, E), 1)
    na = jnp.sum(used)
    ai = (lane * na) // E
    sel = (jnp.broadcast_to(ai, (E, E)) == erow).astype(jnp.float32)
    clean = jnp.where(order_raw == BIG, 0, order_raw).astype(jnp.float32)
    ordf = jax.lax.dot_general(clean, sel, (((1,), (0,)), ((), ())),
                               preferred_element_type=jnp.float32)
    ord_ref[...] = ordf.astype(jnp.int32)


def _moe_body(ord_ref, x_ref, t_ref, c_ref,
              w1a_ref, w2a_ref, w1b_ref, w2b_ref, b1_ref, b2_ref,
              out_ref, s_ref):
    e = pl.program_id(0)

    @pl.when(e == 0)
    def _init():
        si = jax.lax.broadcasted_iota(jnp.int32, (I2, I2 // 2), 0)
        sj = jax.lax.broadcasted_iota(jnp.int32, (I2, I2 // 2), 1)
        s_ref[...] = (si == 2 * sj).astype(jnp.bfloat16)
        out_ref[...] = x_ref[...]

    t = t_ref[...].astype(jnp.bfloat16)
    s = s_ref[...]
    cmat = c_ref[...]
    ecol = jax.lax.broadcasted_iota(jnp.int32, (T, E), 1)
    ords = [ord_ref[EPG * e + p] for p in range(EPG)]
    brow = jax.lax.broadcasted_iota(jnp.int32, (8, E), 0)
    blane = jax.lax.broadcasted_iota(jnp.int32, (8, E), 1)
    oh = ((brow == 0) & (blane == ords[0])) | ((brow == 1) & (blane == ords[1]))
    ohf = oh.astype(jnp.float32)
    b1sel = jax.lax.dot_general(ohf, b1_ref[...], (((1,), (0,)), ((), ())),
                                preferred_element_type=jnp.float32)
    b2sel = jax.lax.dot_general(ohf, b2_ref[...], (((1,), (0,)), ((), ())),
                                preferred_element_type=jnp.float32)
    refs = ((w1a_ref, w2a_ref), (w1b_ref, w2b_ref))
    total = jnp.zeros((T, D), jnp.float32)
    for p in range(EPG):
        w1_ref, w2_ref = refs[p]
        slot = EPG * e + p
        ordv = ords[p]
        prev = ord_ref[jnp.maximum(slot - 1, 0)]
        valid = jnp.where((slot == 0) | (ordv != prev), 1.0, 0.0)
        h = jax.lax.dot_general(t, w1_ref[0].astype(jnp.bfloat16), _DN,
                                preferred_element_type=jnp.float32)
        h = h + b1sel[p:p + 1, :]
        glu = jnp.minimum(h, LIMIT)
        act = glu * jax.nn.sigmoid(ALPHA * glu)
        lin = jnp.clip(h, -LIMIT, LIMIT) + 1.0
        lin_shift = jnp.concatenate([lin[:, 1:], lin[:, :1]], axis=1)
        prod = act * lin_shift
        y = jax.lax.dot_general(prod.astype(jnp.bfloat16), s,
                                (((1,), (0,)), ((), ())),
                                preferred_element_type=jnp.float32)
        o = jax.lax.dot_general(y.astype(jnp.bfloat16),
                                w2_ref[0].astype(jnp.bfloat16), _DN,
                                preferred_element_type=jnp.float32)
        coef = jnp.sum(jnp.where(ecol == ordv, cmat, 0.0),
                       axis=1, keepdims=True) * valid
        total = total + (o + b2sel[p:p + 1, :]) * coef
    out_ref[...] += total


@functools.partial(jax.jit, static_argnames=("interpret",))
def kernel(x, norm_w, gate_w, gate_b, mlp1_w, mlp1_b, mlp2_w, mlp2_b,
           interpret=False):
    t, cmat, order = pl.pallas_call(
        _gate_body,
        in_specs=[
            pl.BlockSpec((T, D), lambda: (0, 0)),
            pl.BlockSpec((1, D), lambda: (0, 0)),
            pl.BlockSpec((E, D), lambda: (0, 0)),
            pl.BlockSpec((1, E), lambda: (0, 0)),
        ],
        out_specs=[
            pl.BlockSpec((T, D), lambda: (0, 0)),
            pl.BlockSpec((T, E), lambda: (0, 0)),
            pl.BlockSpec((1, E), lambda: (0, 0)),
        ],
        out_shape=[
            jax.ShapeDtypeStruct((T, D), jnp.float32),
            jax.ShapeDtypeStruct((T, E), jnp.float32),
            jax.ShapeDtypeStruct((1, E), jnp.int32),
        ],
        interpret=interpret,
    )(x, norm_w.reshape(1, D), gate_w, gate_b.reshape(1, E))

    grid_spec = pltpu.PrefetchScalarGridSpec(
        num_scalar_prefetch=1,
        grid=(E // EPG,),
        in_specs=[
            pl.BlockSpec((T, D), lambda e, o: (0, 0)),
            pl.BlockSpec((T, D), lambda e, o: (0, 0)),
            pl.BlockSpec((T, E), lambda e, o: (0, 0)),
            pl.BlockSpec((1, I2, D), lambda e, o: (o[EPG * e], 0, 0)),
            pl.BlockSpec((1, D, I2 // 2), lambda e, o: (o[EPG * e], 0, 0)),
            pl.BlockSpec((1, I2, D), lambda e, o: (o[EPG * e + 1], 0, 0)),
            pl.BlockSpec((1, D, I2 // 2), lambda e, o: (o[EPG * e + 1], 0, 0)),
            pl.BlockSpec((E, I2), lambda e, o: (0, 0)),
            pl.BlockSpec((E, D), lambda e, o: (0, 0)),
        ],
        out_specs=pl.BlockSpec((T, D), lambda e, o: (0, 0)),
        scratch_shapes=[
            pltpu.VMEM((I2, I2 // 2), jnp.bfloat16),
        ],
    )
    out = pl.pallas_call(
        _moe_body,
        grid_spec=grid_spec,
        out_shape=jax.ShapeDtypeStruct((T, D), jnp.float32),
        interpret=interpret,
    )(order.reshape(E), x, t, cmat,
      mlp1_w, mlp2_w, mlp1_w, mlp2_w, mlp1_b, mlp2_b)
    return out


# speedup vs baseline: 1.0673x; 1.0673x over previous
import functools

import jax
import jax.numpy as jnp
from jax.experimental import pallas as pl
from jax.experimental.pallas import tpu as pltpu

T = 64
D = 768
E = 64
I2 = 1536
EPG = 2
LIMIT = 7.0
ALPHA = 1.702
BIG = 1 << 20

_DN = (((1,), (1,)), ((), ()))


def _gate_body(x_ref, nw_ref, gw_ref, gb_ref, t_ref, c_ref, ord_ref):
    xv = x_ref[...]
    ms = jnp.mean(xv * xv, axis=1, keepdims=True)
    eps = jnp.finfo(jnp.float32).eps
    t = xv * jax.lax.rsqrt(ms + eps) * nw_ref[...]
    t_ref[...] = t
    g = jax.lax.dot_general(t, gw_ref[...], _DN,
                            preferred_element_type=jnp.float32)
    g = g + gb_ref[...]
    col = jax.lax.broadcasted_iota(jnp.int32, (T, E), 1)
    m1 = jnp.max(g, axis=1, keepdims=True)
    i1 = jnp.min(jnp.where(g == m1, col, E), axis=1, keepdims=True)
    oh1 = col == i1
    gm = jnp.where(oh1, -jnp.inf, g)
    m2 = jnp.max(gm, axis=1, keepdims=True)
    i2 = jnp.min(jnp.where(gm == m2, col, E), axis=1, keepdims=True)
    oh2 = col == i2
    p1 = jax.nn.sigmoid(m1 - m2)
    c_ref[...] = jnp.where(oh1, p1, 0.0) + jnp.where(oh2, 1.0 - p1, 0.0)

    used = jnp.max((oh1 | oh2).astype(jnp.int32), axis=0, keepdims=True)
    erow = jax.lax.broadcasted_iota(jnp.int32, (E, E), 0)
    icol = jax.lax.broadcasted_iota(jnp.int32, (E, E), 1)
    lt = (erow <= icol).astype(jnp.float32)
    csum = jax.lax.dot_general(used.astype(jnp.float32), lt,
                               (((1,), (0,)), ((), ())),
                               preferred_element_type=jnp.float32)
    csum_i = csum.astype(jnp.int32)
    used_e = jnp.broadcast_to(used.reshape(E, 1), (E, E)) > 0
    csum_e = jnp.broadcast_to(csum_i.reshape(E, 1), (E, E))
    cond = used_e & (csum_e == icol + 1)
    order_raw = jnp.min(jnp.where(cond, erow, BIG), axis=0, keepdims=True)
    lane = jax.lax.broadcasted_iota(jnp.int32, (1, E), 1)
    na = jnp.sum(used)
    ca = (na + 1) // 2
    cb = na // 2
    j = lane // 2
    nsteps = E // EPG
    ai = jnp.where(lane % 2 == 0, (j * ca) // nsteps,
                   ca + (j * cb) // nsteps)
    sel = (jnp.broadcast_to(ai, (E, E)) == erow).astype(jnp.float32)
    clean = jnp.where(order_raw == BIG, 0, order_raw).astype(jnp.float32)
    ordf = jax.lax.dot_general(clean, sel, (((1,), (0,)), ((), ())),
                               preferred_element_type=jnp.float32)
    ord_ref[...] = ordf.astype(jnp.int32)


def _moe_body(ord_ref, x_ref, t_ref, c_ref,
              w1a_ref, w2a_ref, w1b_ref, w2b_ref, b1_ref, b2_ref,
              out_ref, s_ref):
    e = pl.program_id(0)

    @pl.when(e == 0)
    def _init():
        si = jax.lax.broadcasted_iota(jnp.int32, (I2, I2 // 2), 0)
        sj = jax.lax.broadcasted_iota(jnp.int32, (I2, I2 // 2), 1)
        s_ref[...] = (si == 2 * sj).astype(jnp.bfloat16)
        out_ref[...] = x_ref[...]

    t = t_ref[...].astype(jnp.bfloat16)
    s = s_ref[...]
    cmat = c_ref[...]
    ecol = jax.lax.broadcasted_iota(jnp.int32, (T, E), 1)
    ords = [ord_ref[EPG * e + p] for p in range(EPG)]
    brow = jax.lax.broadcasted_iota(jnp.int32, (8, E), 0)
    blane = jax.lax.broadcasted_iota(jnp.int32, (8, E), 1)
    oh = ((brow == 0) & (blane == ords[0])) | ((brow == 1) & (blane == ords[1]))
    ohf = oh.astype(jnp.float32)
    b1sel = jax.lax.dot_general(ohf, b1_ref[...], (((1,), (0,)), ((), ())),
                                preferred_element_type=jnp.float32)
    b2sel = jax.lax.dot_general(ohf, b2_ref[...], (((1,), (0,)), ((), ())),
                                preferred_element_type=jnp.float32)
    refs = ((w1a_ref, w2a_ref), (w1b_ref, w2b_ref))
    total = jnp.zeros((T, D), jnp.float32)
    for p in range(EPG):
        w1_ref, w2_ref = refs[p]
        slot = EPG * e + p
        ordv = ords[p]
        prev = ord_ref[jnp.maximum(slot - EPG, 0)]
        valid = jnp.where((slot < EPG) | (ordv != prev), 1.0, 0.0)
        h = jax.lax.dot_general(t, w1_ref[0].astype(jnp.bfloat16), _DN,
                                preferred_element_type=jnp.float32)
        h = h + b1sel[p:p + 1, :]
        glu = jnp.minimum(h, LIMIT)
        act = glu * jax.nn.sigmoid(ALPHA * glu)
        lin = jnp.clip(h, -LIMIT, LIMIT) + 1.0
        lin_shift = jnp.concatenate([lin[:, 1:], lin[:, :1]], axis=1)
        prod = act * lin_shift
        y = jax.lax.dot_general(prod.astype(jnp.bfloat16), s,
                                (((1,), (0,)), ((), ())),
                                preferred_element_type=jnp.float32)
        o = jax.lax.dot_general(y.astype(jnp.bfloat16),
                                w2_ref[0].astype(jnp.bfloat16), _DN,
                                preferred_element_type=jnp.float32)
        coef = jnp.sum(jnp.where(ecol == ordv, cmat, 0.0),
                       axis=1, keepdims=True) * valid
        total = total + (o + b2sel[p:p + 1, :]) * coef
    out_ref[...] += total


@functools.partial(jax.jit, static_argnames=("interpret",))
def kernel(x, norm_w, gate_w, gate_b, mlp1_w, mlp1_b, mlp2_w, mlp2_b,
           interpret=False):
    t, cmat, order = pl.pallas_call(
        _gate_body,
        in_specs=[
            pl.BlockSpec((T, D), lambda: (0, 0)),
            pl.BlockSpec((1, D), lambda: (0, 0)),
            pl.BlockSpec((E, D), lambda: (0, 0)),
            pl.BlockSpec((1, E), lambda: (0, 0)),
        ],
        out_specs=[
            pl.BlockSpec((T, D), lambda: (0, 0)),
            pl.BlockSpec((T, E), lambda: (0, 0)),
            pl.BlockSpec((1, E), lambda: (0, 0)),
        ],
        out_shape=[
            jax.ShapeDtypeStruct((T, D), jnp.float32),
            jax.ShapeDtypeStruct((T, E), jnp.float32),
            jax.ShapeDtypeStruct((1, E), jnp.int32),
        ],
        interpret=interpret,
    )(x, norm_w.reshape(1, D), gate_w, gate_b.reshape(1, E))

    grid_spec = pltpu.PrefetchScalarGridSpec(
        num_scalar_prefetch=1,
        grid=(E // EPG,),
        in_specs=[
            pl.BlockSpec((T, D), lambda e, o: (0, 0)),
            pl.BlockSpec((T, D), lambda e, o: (0, 0)),
            pl.BlockSpec((T, E), lambda e, o: (0, 0)),
            pl.BlockSpec((1, I2, D), lambda e, o: (o[EPG * e], 0, 0)),
            pl.BlockSpec((1, D, I2 // 2), lambda e, o: (o[EPG * e], 0, 0)),
            pl.BlockSpec((1, I2, D), lambda e, o: (o[EPG * e + 1], 0, 0)),
            pl.BlockSpec((1, D, I2 // 2), lambda e, o: (o[EPG * e + 1], 0, 0)),
            pl.BlockSpec((E, I2), lambda e, o: (0, 0)),
            pl.BlockSpec((E, D), lambda e, o: (0, 0)),
        ],
        out_specs=pl.BlockSpec((T, D), lambda e, o: (0, 0)),
        scratch_shapes=[
            pltpu.VMEM((I2, I2 // 2), jnp.bfloat16),
        ],
    )
    out = pl.pallas_call(
        _moe_body,
        grid_spec=grid_spec,
        out_shape=jax.ShapeDtypeStruct((T, D), jnp.float32),
        interpret=interpret,
    )(order.reshape(E), x, t, cmat,
      mlp1_w, mlp2_w, mlp1_w, mlp2_w, mlp1_b, mlp2_b)
    return out
